# baseline (speedup 1.0000x reference)
import functools

import jax
import jax.numpy as jnp
from jax import lax
from jax.experimental import pallas as pl
from jax.experimental.pallas import tpu as pltpu
from jax.experimental.pallas import tpu_sc as plsc

N = 10000
C = 128
E = 320000
HALF = N // 2
ACC_R = 5248
DUMMY = HALF
B = 640
NB = E // B
GB = 128
C2 = 2 * C


def _sc_spmm(x2, rows, cols, vr, vi):
    mesh = plsc.VectorSubcoreMesh(core_axis_name="c", subcore_axis_name="s")

    @functools.partial(
        pl.kernel,
        mesh=mesh,
        compiler_params=pltpu.CompilerParams(
            needs_layout_passes=False, use_tc_tiling_on_sc=False,
            disable_bounds_checks=True),
        out_type=jax.ShapeDtypeStruct((2 * C2 * ACC_R,), jnp.float32),
        scratch_types=[
            pltpu.VMEM((B,), jnp.int32),
            pltpu.VMEM((B,), jnp.int32),
            pltpu.VMEM((B,), jnp.int32),
            pltpu.VMEM((B,), jnp.int32),
            pltpu.VMEM((B,), jnp.float32),
            pltpu.VMEM((B,), jnp.float32),
            pltpu.VMEM((B,), jnp.float32),
            pltpu.VMEM((B,), jnp.float32),
            pltpu.VMEM((B,), jnp.int32),
            pltpu.VMEM((B,), jnp.int32),
            pltpu.VMEM((B, 16), jnp.float32),
            pltpu.VMEM((B, 16), jnp.float32),
            pltpu.VMEM((16 * ACC_R,), jnp.float32),
            pltpu.SemaphoreType.DMA,
            pltpu.SemaphoreType.DMA,
            pltpu.SemaphoreType.DMA,
            pltpu.SemaphoreType.DMA,
        ],
    )
    def k(x_hbm, rows_hbm, cols_hbm, vr_hbm, vi_hbm, a_out,
          rows_v0, rows_v1, cols_v0, cols_v1, vr_v0, vr_v1, vi_v0, vi_v1,
          colg_v0, colg_v1, xbuf0, xbuf1, acc_t, sm0, sm1, sg0, sg1):
        c = lax.axis_index("c")
        s = lax.axis_index("s")
        row_base = c * HALF
        zero16 = jnp.zeros((16,), jnp.float32)
        iota = lax.iota(jnp.int32, 16)
        xg = x_hbm

        def fire_meta(b, bufs, sem):
            rv, cv, vrv, viv = bufs
            sl = pl.ds(b * B, B)
            pltpu.async_copy(rows_hbm.at[sl], rv, sem)
            pltpu.async_copy(cols_hbm.at[sl], cv, sem)
            pltpu.async_copy(vr_hbm.at[sl], vrv, sem)
            pltpu.async_copy(vi_hbm.at[sl], viv, sem)

        def wait_meta(bufs, sem):
            rv, cv, vrv, viv = bufs
            sl = pl.ds(0, B)
            pltpu.make_async_copy(rows_hbm.at[sl], rv, sem).wait()
            pltpu.make_async_copy(cols_hbm.at[sl], cv, sem).wait()
            pltpu.make_async_copy(vr_hbm.at[sl], vrv, sem).wait()
            pltpu.make_async_copy(vi_hbm.at[sl], viv, sem).wait()

        def prep(bufs, cg):
            cv = bufs[1]
            @plsc.parallel_loop(0, B // 16, unroll=2)
            def _(kk):
                cols16 = cv[pl.ds(kk * 16, 16)]
                cg[pl.ds(kk * 16, 16)] = cols16 * 16 + s

        def fire_gathers(cg, xb, sem):
            for q in range(B // GB):
                pltpu.async_copy(xg.at[cg.at[pl.ds(q * GB, GB)]],
                                 xb.at[pl.ds(q * GB, GB)], sem)

        def wait_gathers(cg, xb, sem):
            for q in range(B // GB):
                pltpu.make_async_copy(xg.at[cg.at[pl.ds(q * GB, GB)]],
                                      xb.at[pl.ds(q * GB, GB)], sem).wait()

        def compute(bufs, xb):
            rv, _, vrv, viv = bufs

            @plsc.parallel_loop(0, B // 16, unroll=2)
            def grp_body(kk):
                eidx = kk * 16 + iota
                rows16 = rv[pl.ds(kk * 16, 16)]
                vrr = vrv[pl.ds(kk * 16, 16)]
                vii = viv[pl.ds(kk * 16, 16)]
                local = rows16 - row_base
                ok = (local >= 0) & (local < HALF)
                lr = jnp.where(ok, local, DUMMY)
                for m in range(8):
                    xr = plsc.load_gather(xb, [eidx, jnp.full((16,), 2 * m, jnp.int32)])
                    xi = plsc.load_gather(xb, [eidx, jnp.full((16,), 2 * m + 1, jnp.int32)])
                    orv = vrr * xr - vii * xi
                    oiv = vii * xr + vrr * xi
                    plsc.addupdate_scatter(
                        acc_t, [lr + (2 * m) * ACC_R], orv)
                    plsc.addupdate_scatter(
                        acc_t, [lr + (2 * m + 1) * ACC_R], oiv)

        bufs0 = (rows_v0, cols_v0, vr_v0, vi_v0)
        bufs1 = (rows_v1, cols_v1, vr_v1, vi_v1)

        fire_meta(0, bufs0, sm0)
        fire_meta(1, bufs1, sm1)

        def zrow(r, carry):
            acc_t[pl.ds(r * 16, 16)] = zero16
            return carry
        lax.fori_loop(0, 16 * ACC_R // 16, zrow, 0)

        wait_meta(bufs0, sm0)
        prep(bufs0, colg_v0)
        fire_gathers(colg_v0, xbuf0, sg0)

        def pair_body(t, carry):
            b0 = 2 * t
            wait_meta(bufs1, sm1)
            prep(bufs1, colg_v1)
            fire_gathers(colg_v1, xbuf1, sg1)

            wait_gathers(colg_v0, xbuf0, sg0)
            compute(bufs0, xbuf0)

            @pl.when(b0 + 2 < NB)
            def _():
                fire_meta(b0 + 2, bufs0, sm0)

            wait_gathers(colg_v1, xbuf1, sg1)
            compute(bufs1, xbuf1)

            @pl.when(b0 + 3 < NB)
            def _():
                fire_meta(b0 + 3, bufs1, sm1)

            @pl.when(b0 + 2 < NB)
            def _():
                wait_meta(bufs0, sm0)
                prep(bufs0, colg_v0)
                fire_gathers(colg_v0, xbuf0, sg0)
            return carry
        lax.fori_loop(0, NB // 2, pair_body, 0)

        for ch in range(16):
            u = c * C2 + 16 * s + ch
            pltpu.sync_copy(acc_t.at[pl.ds(ch * ACC_R, ACC_R)],
                            a_out.at[pl.ds(u * ACC_R, ACC_R)])

    return k(x2, rows, cols, vr, vi)


def _tc_body(a_ref, w_ref, xr_ref, xi_ref, or_ref, oi_ref):
    a = a_ref[0]
    res = lax.dot_general(a, w_ref[...], (((0,), (0,)), ((), ())),
                          preferred_element_type=jnp.float32)
    or_ref[...] = res[:HALF, :C] + xr_ref[...]
    oi_ref[...] = res[:HALF, C:] + xi_ref[...]


def _tc_matmul(a_full, x_real, x_imag, w_big):
    return pl.pallas_call(
        _tc_body,
        grid=(2,),
        in_specs=[
            pl.BlockSpec((1, C2, ACC_R), lambda i: (i, 0, 0)),
            pl.BlockSpec((C2, C2), lambda i: (0, 0)),
            pl.BlockSpec((HALF, C), lambda i: (i, 0)),
            pl.BlockSpec((HALF, C), lambda i: (i, 0)),
        ],
        out_specs=[
            pl.BlockSpec((HALF, C), lambda i: (i, 0)),
            pl.BlockSpec((HALF, C), lambda i: (i, 0)),
        ],
        out_shape=[
            jax.ShapeDtypeStruct((N, C), jnp.float32),
            jax.ShapeDtypeStruct((N, C), jnp.float32),
        ],
    )(a_full, w_big, x_real, x_imag)


@jax.jit
def kernel(X_real, X_imag, edge_index, L_real_vals, L_imag_vals, weight):
    x2 = jnp.stack([X_real, X_imag], axis=2).reshape(N * 16, 16)
    rows = edge_index[0]
    cols = edge_index[1]
    ch = jnp.arange(C)
    u_r = (ch // 8) * 16 + (ch % 8) * 2
    w_big = jnp.zeros((C2, C2), jnp.float32)
    w_big = w_big.at[u_r, :C].set(weight).at[u_r + 1, C:].set(weight)

    a1d = _sc_spmm(x2, rows, cols, L_real_vals, L_imag_vals)
    a_full = a1d.reshape(2, C2, ACC_R)
    return _tc_matmul(a_full, X_real, X_imag, w_big)


# speedup vs baseline: 1.7044x; 1.7044x over previous
import functools

import jax
import jax.numpy as jnp
from jax import lax
from jax.experimental import pallas as pl
from jax.experimental.pallas import tpu as pltpu
from jax.experimental.pallas import tpu_sc as plsc

N = 10000
C = 128
E = 320000
HALF = N // 2
ACC_R = 5248
DUMMY = HALF
B = 640
NB = E // B
GB = 128
C2 = 2 * C


def _sc_spmm(x2, rows, cols, vr, vi):
    mesh = plsc.VectorSubcoreMesh(core_axis_name="c", subcore_axis_name="s")

    @functools.partial(
        pl.kernel,
        mesh=mesh,
        compiler_params=pltpu.CompilerParams(
            needs_layout_passes=False, use_tc_tiling_on_sc=False,
            disable_bounds_checks=True),
        out_type=jax.ShapeDtypeStruct((2 * C2 * ACC_R,), jnp.float32),
        scratch_types=[
            pltpu.VMEM((B,), jnp.int32),
            pltpu.VMEM((B,), jnp.int32),
            pltpu.VMEM((B,), jnp.int32),
            pltpu.VMEM((B,), jnp.int32),
            pltpu.VMEM((B,), jnp.float32),
            pltpu.VMEM((B,), jnp.float32),
            pltpu.VMEM((B,), jnp.float32),
            pltpu.VMEM((B,), jnp.float32),
            pltpu.VMEM((B,), jnp.int32),
            pltpu.VMEM((B,), jnp.int32),
            pltpu.VMEM((B, 16), jnp.float32),
            pltpu.VMEM((B, 16), jnp.float32),
            pltpu.VMEM((16, ACC_R), jnp.float32),
            pltpu.SemaphoreType.DMA,
            pltpu.SemaphoreType.DMA,
            pltpu.SemaphoreType.DMA,
            pltpu.SemaphoreType.DMA,
        ],
    )
    def k(x_hbm, rows_hbm, cols_hbm, vr_hbm, vi_hbm, a_out,
          rows_v0, rows_v1, cols_v0, cols_v1, vr_v0, vr_v1, vi_v0, vi_v1,
          colg_v0, colg_v1, xbuf0, xbuf1, acc_t, sm0, sm1, sg0, sg1):
        c = lax.axis_index("c")
        s = lax.axis_index("s")
        row_base = c * HALF
        zero16 = jnp.zeros((16,), jnp.float32)
        iota = lax.iota(jnp.int32, 16)
        xg = x_hbm

        def fire_meta(b, bufs, sem):
            rv, cv, vrv, viv = bufs
            sl = pl.ds(b * B, B)
            pltpu.async_copy(rows_hbm.at[sl], rv, sem)
            pltpu.async_copy(cols_hbm.at[sl], cv, sem)
            pltpu.async_copy(vr_hbm.at[sl], vrv, sem)
            pltpu.async_copy(vi_hbm.at[sl], viv, sem)

        def wait_meta(bufs, sem):
            rv, cv, vrv, viv = bufs
            sl = pl.ds(0, B)
            pltpu.make_async_copy(rows_hbm.at[sl], rv, sem).wait()
            pltpu.make_async_copy(cols_hbm.at[sl], cv, sem).wait()
            pltpu.make_async_copy(vr_hbm.at[sl], vrv, sem).wait()
            pltpu.make_async_copy(vi_hbm.at[sl], viv, sem).wait()

        def prep(bufs, cg):
            cv = bufs[1]
            @plsc.parallel_loop(0, B // 16, unroll=2)
            def _(kk):
                cols16 = cv[pl.ds(kk * 16, 16)]
                cg[pl.ds(kk * 16, 16)] = cols16 * 16 + s

        def fire_gathers(cg, xb, sem):
            for q in range(B // GB):
                pltpu.async_copy(xg.at[cg.at[pl.ds(q * GB, GB)]],
                                 xb.at[pl.ds(q * GB, GB)], sem)

        def wait_gathers(cg, xb, sem):
            for q in range(B // GB):
                pltpu.make_async_copy(xg.at[cg.at[pl.ds(q * GB, GB)]],
                                      xb.at[pl.ds(q * GB, GB)], sem).wait()

        def compute(bufs, xb):
            rv, _, vrv, viv = bufs

            @plsc.parallel_loop(0, B // 16, unroll=4)
            def grp_body(kk):
                eidx = kk * 16 + iota
                rows16 = rv[pl.ds(kk * 16, 16)]
                vrr = vrv[pl.ds(kk * 16, 16)]
                vii = viv[pl.ds(kk * 16, 16)]
                local = rows16 - row_base
                ok = (local >= 0) & (local < HALF)
                lr = jnp.where(ok, local, DUMMY)
                for m in range(8):
                    xr = plsc.load_gather(xb, [eidx, jnp.full((16,), 2 * m, jnp.int32)])
                    xi = plsc.load_gather(xb, [eidx, jnp.full((16,), 2 * m + 1, jnp.int32)])
                    orv = vrr * xr - vii * xi
                    oiv = vii * xr + vrr * xi
                    plsc.addupdate_scatter(
                        acc_t, [jnp.full((16,), 2 * m, jnp.int32), lr], orv)
                    plsc.addupdate_scatter(
                        acc_t, [jnp.full((16,), 2 * m + 1, jnp.int32), lr], oiv)

        bufs0 = (rows_v0, cols_v0, vr_v0, vi_v0)
        bufs1 = (rows_v1, cols_v1, vr_v1, vi_v1)

        fire_meta(0, bufs0, sm0)
        fire_meta(1, bufs1, sm1)

        def zrow(r, carry):
            for ch in range(16):
                acc_t[ch, pl.ds(r * 16, 16)] = zero16
            return carry
        lax.fori_loop(0, ACC_R // 16, zrow, 0)

        wait_meta(bufs0, sm0)
        prep(bufs0, colg_v0)
        fire_gathers(colg_v0, xbuf0, sg0)

        def pair_body(t, carry):
            b0 = 2 * t
            wait_meta(bufs1, sm1)
            prep(bufs1, colg_v1)
            fire_gathers(colg_v1, xbuf1, sg1)

            wait_gathers(colg_v0, xbuf0, sg0)
            compute(bufs0, xbuf0)

            @pl.when(b0 + 2 < NB)
            def _():
                fire_meta(b0 + 2, bufs0, sm0)

            wait_gathers(colg_v1, xbuf1, sg1)
            compute(bufs1, xbuf1)

            @pl.when(b0 + 3 < NB)
            def _():
                fire_meta(b0 + 3, bufs1, sm1)

            @pl.when(b0 + 2 < NB)
            def _():
                wait_meta(bufs0, sm0)
                prep(bufs0, colg_v0)
                fire_gathers(colg_v0, xbuf0, sg0)
            return carry
        lax.fori_loop(0, NB // 2, pair_body, 0)

        for ch in range(16):
            u = c * C2 + 16 * s + ch
            pltpu.sync_copy(acc_t.at[ch],
                            a_out.at[pl.ds(u * ACC_R, ACC_R)])

    return k(x2, rows, cols, vr, vi)


def _src_perm_f32(rows_are_src):
    r_io = lax.broadcasted_iota(jnp.int32, (C2, C2), 0)
    c_io = lax.broadcasted_iota(jnp.int32, (C2, C2), 1)
    v = r_io if rows_are_src else c_io
    other = c_io if rows_are_src else r_io
    src = (v % 2) * C + 8 * (v // 16) + (v % 16) // 2
    return jnp.where(other == src, 1.0, 0.0).astype(jnp.float32)


def _tc_pre_body(xr_ref, xi_ref, o_ref):
    cat = jnp.concatenate([xr_ref[...], xi_ref[...]], axis=1)
    p_int = _src_perm_f32(rows_are_src=False)
    o_ref[...] = jnp.dot(cat, p_int, preferred_element_type=jnp.float32)


def _tc_interleave(x_real, x_imag):
    rb = 2000
    return pl.pallas_call(
        _tc_pre_body,
        grid=(N // rb,),
        in_specs=[
            pl.BlockSpec((rb, C), lambda i: (i, 0)),
            pl.BlockSpec((rb, C), lambda i: (i, 0)),
        ],
        out_specs=pl.BlockSpec((rb, C2), lambda i: (i, 0)),
        out_shape=jax.ShapeDtypeStruct((N, C2), jnp.float32),
    )(x_real, x_imag)


def _tc_body(a_ref, w_ref, xr_ref, xi_ref, or_ref, oi_ref):
    a = a_ref[0]
    w = w_ref[...]
    zc = jnp.zeros((C, C), jnp.float32)
    wblock = jnp.concatenate(
        [jnp.concatenate([w, zc], axis=1),
         jnp.concatenate([zc, w], axis=1)], axis=0)
    m_perm = _src_perm_f32(rows_are_src=True)
    w_big = jnp.dot(m_perm, wblock, preferred_element_type=jnp.float32)
    res = lax.dot_general(a, w_big, (((0,), (0,)), ((), ())),
                          preferred_element_type=jnp.float32)
    or_ref[...] = res[:HALF, :C] + xr_ref[...]
    oi_ref[...] = res[:HALF, C:] + xi_ref[...]


def _tc_matmul(a_full, x_real, x_imag, weight):
    return pl.pallas_call(
        _tc_body,
        grid=(2,),
        in_specs=[
            pl.BlockSpec((1, C2, ACC_R), lambda i: (i, 0, 0)),
            pl.BlockSpec((C, C), lambda i: (0, 0)),
            pl.BlockSpec((HALF, C), lambda i: (i, 0)),
            pl.BlockSpec((HALF, C), lambda i: (i, 0)),
        ],
        out_specs=[
            pl.BlockSpec((HALF, C), lambda i: (i, 0)),
            pl.BlockSpec((HALF, C), lambda i: (i, 0)),
        ],
        out_shape=[
            jax.ShapeDtypeStruct((N, C), jnp.float32),
            jax.ShapeDtypeStruct((N, C), jnp.float32),
        ],
    )(a_full, weight, x_real, x_imag)


@jax.jit
def kernel(X_real, X_imag, edge_index, L_real_vals, L_imag_vals, weight):
    x2 = _tc_interleave(X_real, X_imag).reshape(N * 16, 16)
    rows = edge_index[0]
    cols = edge_index[1]

    a1d = _sc_spmm(x2, rows, cols, L_real_vals, L_imag_vals)
    a_full = a1d.reshape(2, C2, ACC_R)
    return _tc_matmul(a_full, X_real, X_imag, weight)


# speedup vs baseline: 1.8817x; 1.1040x over previous
import functools

import jax
import jax.numpy as jnp
from jax import lax
from jax.experimental import pallas as pl
from jax.experimental.pallas import tpu as pltpu
from jax.experimental.pallas import tpu_sc as plsc

N = 10000
C = 128
E = 320000
HALF = N // 2
ACC_R = 5248
DUMMY = HALF
B = 640
NB = E // B
GB = 128
C2 = 2 * C


def _sc_spmm(x2, rows, cols, vr, vi):
    mesh = plsc.VectorSubcoreMesh(core_axis_name="c", subcore_axis_name="s")

    @functools.partial(
        pl.kernel,
        mesh=mesh,
        compiler_params=pltpu.CompilerParams(
            needs_layout_passes=False, use_tc_tiling_on_sc=False,
            disable_bounds_checks=True),
        out_type=jax.ShapeDtypeStruct((2 * C2 * ACC_R,), jnp.float32),
        scratch_types=[
            pltpu.VMEM((B,), jnp.int32),
            pltpu.VMEM((B,), jnp.int32),
            pltpu.VMEM((B,), jnp.int32),
            pltpu.VMEM((B,), jnp.int32),
            pltpu.VMEM((B,), jnp.float32),
            pltpu.VMEM((B,), jnp.float32),
            pltpu.VMEM((B,), jnp.float32),
            pltpu.VMEM((B,), jnp.float32),
            pltpu.VMEM((B,), jnp.int32),
            pltpu.VMEM((B,), jnp.int32),
            pltpu.VMEM((B, 16), jnp.float32),
            pltpu.VMEM((B, 16), jnp.float32),
            pltpu.VMEM((16, ACC_R), jnp.float32),
            pltpu.SemaphoreType.DMA,
            pltpu.SemaphoreType.DMA,
            pltpu.SemaphoreType.DMA,
            pltpu.SemaphoreType.DMA,
        ],
    )
    def k(x_hbm, rows_hbm, cols_hbm, vr_hbm, vi_hbm, a_out,
          rows_v0, rows_v1, cols_v0, cols_v1, vr_v0, vr_v1, vi_v0, vi_v1,
          colg_v0, colg_v1, xbuf0, xbuf1, acc_t, sm0, sm1, sg0, sg1):
        c = lax.axis_index("c")
        s = lax.axis_index("s")
        row_base = c * HALF
        zero16 = jnp.zeros((16,), jnp.float32)
        iota = lax.iota(jnp.int32, 16)
        xg = x_hbm

        def fire_meta(b, bufs, sem):
            rv, cv, vrv, viv = bufs
            sl = pl.ds(b * B, B)
            pltpu.async_copy(rows_hbm.at[sl], rv, sem)
            pltpu.async_copy(cols_hbm.at[sl], cv, sem)
            pltpu.async_copy(vr_hbm.at[sl], vrv, sem)
            pltpu.async_copy(vi_hbm.at[sl], viv, sem)

        def wait_meta(bufs, sem):
            rv, cv, vrv, viv = bufs
            sl = pl.ds(0, B)
            pltpu.make_async_copy(rows_hbm.at[sl], rv, sem).wait()
            pltpu.make_async_copy(cols_hbm.at[sl], cv, sem).wait()
            pltpu.make_async_copy(vr_hbm.at[sl], vrv, sem).wait()
            pltpu.make_async_copy(vi_hbm.at[sl], viv, sem).wait()

        def prep(bufs, cg):
            cv = bufs[1]
            @plsc.parallel_loop(0, B // 16, unroll=2)
            def _(kk):
                cols16 = cv[pl.ds(kk * 16, 16)]
                cg[pl.ds(kk * 16, 16)] = cols16 * 16 + s

        def fire_gathers(cg, xb, sem):
            for q in range(B // GB):
                pltpu.async_copy(xg.at[cg.at[pl.ds(q * GB, GB)]],
                                 xb.at[pl.ds(q * GB, GB)], sem)

        def wait_gathers(cg, xb, sem):
            for q in range(B // GB):
                pltpu.make_async_copy(xg.at[cg.at[pl.ds(q * GB, GB)]],
                                      xb.at[pl.ds(q * GB, GB)], sem).wait()

        def compute(bufs, xb):
            rv, _, vrv, viv = bufs

            @plsc.parallel_loop(0, B // 16, unroll=2)
            def grp_body(kk):
                eidx = kk * 16 + iota
                rows16 = rv[pl.ds(kk * 16, 16)]
                vrr = vrv[pl.ds(kk * 16, 16)]
                vii = viv[pl.ds(kk * 16, 16)]
                local = rows16 - row_base
                ok = (local >= 0) & (local < HALF)
                lr = jnp.where(ok, local, DUMMY)
                for m in range(8):
                    xr = plsc.load_gather(xb, [eidx, jnp.full((16,), 2 * m, jnp.int32)])
                    xi = plsc.load_gather(xb, [eidx, jnp.full((16,), 2 * m + 1, jnp.int32)])
                    orv = vrr * xr - vii * xi
                    oiv = vii * xr + vrr * xi
                    plsc.addupdate_scatter(
                        acc_t, [jnp.full((16,), 2 * m, jnp.int32), lr], orv)
                    plsc.addupdate_scatter(
                        acc_t, [jnp.full((16,), 2 * m + 1, jnp.int32), lr], oiv)

        bufs0 = (rows_v0, cols_v0, vr_v0, vi_v0)
        bufs1 = (rows_v1, cols_v1, vr_v1, vi_v1)

        fire_meta(0, bufs0, sm0)
        fire_meta(1, bufs1, sm1)

        def zrow(r, carry):
            for ch in range(16):
                acc_t[ch, pl.ds(r * 16, 16)] = zero16
            return carry
        lax.fori_loop(0, ACC_R // 16, zrow, 0)

        wait_meta(bufs0, sm0)
        prep(bufs0, colg_v0)
        fire_gathers(colg_v0, xbuf0, sg0)

        def pair_body(t, carry):
            b0 = 2 * t
            wait_meta(bufs1, sm1)
            prep(bufs1, colg_v1)
            fire_gathers(colg_v1, xbuf1, sg1)

            wait_gathers(colg_v0, xbuf0, sg0)
            compute(bufs0, xbuf0)

            @pl.when(b0 + 2 < NB)
            def _():
                fire_meta(b0 + 2, bufs0, sm0)

            wait_gathers(colg_v1, xbuf1, sg1)
            compute(bufs1, xbuf1)

            @pl.when(b0 + 3 < NB)
            def _():
                fire_meta(b0 + 3, bufs1, sm1)

            @pl.when(b0 + 2 < NB)
            def _():
                wait_meta(bufs0, sm0)
                prep(bufs0, colg_v0)
                fire_gathers(colg_v0, xbuf0, sg0)
            return carry
        lax.fori_loop(0, NB // 2, pair_body, 0)

        for ch in range(16):
            u = c * C2 + 16 * s + ch
            pltpu.sync_copy(acc_t.at[ch],
                            a_out.at[pl.ds(u * ACC_R, ACC_R)])

    return k(x2, rows, cols, vr, vi)


def _src_perm_f32(rows_are_src):
    r_io = lax.broadcasted_iota(jnp.int32, (C2, C2), 0)
    c_io = lax.broadcasted_iota(jnp.int32, (C2, C2), 1)
    v = r_io if rows_are_src else c_io
    other = c_io if rows_are_src else r_io
    src = (v % 2) * C + 8 * (v // 16) + (v % 16) // 2
    return jnp.where(other == src, 1.0, 0.0).astype(jnp.float32)


def _tc_pre_body(xr_ref, xi_ref, o_ref):
    cat = jnp.concatenate([xr_ref[...], xi_ref[...]], axis=1)
    p_int = _src_perm_f32(rows_are_src=False)
    o_ref[...] = jnp.dot(cat, p_int, preferred_element_type=jnp.float32)


def _tc_interleave(x_real, x_imag):
    rb = 2000
    return pl.pallas_call(
        _tc_pre_body,
        grid=(N // rb,),
        in_specs=[
            pl.BlockSpec((rb, C), lambda i: (i, 0)),
            pl.BlockSpec((rb, C), lambda i: (i, 0)),
        ],
        out_specs=pl.BlockSpec((rb, C2), lambda i: (i, 0)),
        out_shape=jax.ShapeDtypeStruct((N, C2), jnp.float32),
    )(x_real, x_imag)


def _tc_body(a_ref, w_ref, xr_ref, xi_ref, or_ref, oi_ref):
    a = a_ref[0]
    w = w_ref[...]
    zc = jnp.zeros((C, C), jnp.float32)
    wblock = jnp.concatenate(
        [jnp.concatenate([w, zc], axis=1),
         jnp.concatenate([zc, w], axis=1)], axis=0)
    m_perm = _src_perm_f32(rows_are_src=True)
    w_big = jnp.dot(m_perm, wblock, preferred_element_type=jnp.float32)
    res = lax.dot_general(a, w_big, (((0,), (0,)), ((), ())),
                          preferred_element_type=jnp.float32)
    or_ref[...] = res[:HALF, :C] + xr_ref[...]
    oi_ref[...] = res[:HALF, C:] + xi_ref[...]


def _tc_matmul(a_full, x_real, x_imag, weight):
    return pl.pallas_call(
        _tc_body,
        grid=(2,),
        in_specs=[
            pl.BlockSpec((1, C2, ACC_R), lambda i: (i, 0, 0)),
            pl.BlockSpec((C, C), lambda i: (0, 0)),
            pl.BlockSpec((HALF, C), lambda i: (i, 0)),
            pl.BlockSpec((HALF, C), lambda i: (i, 0)),
        ],
        out_specs=[
            pl.BlockSpec((HALF, C), lambda i: (i, 0)),
            pl.BlockSpec((HALF, C), lambda i: (i, 0)),
        ],
        out_shape=[
            jax.ShapeDtypeStruct((N, C), jnp.float32),
            jax.ShapeDtypeStruct((N, C), jnp.float32),
        ],
    )(a_full, weight, x_real, x_imag)


@jax.jit
def kernel(X_real, X_imag, edge_index, L_real_vals, L_imag_vals, weight):
    x2 = _tc_interleave(X_real, X_imag).reshape(N * 16, 16)
    rows = edge_index[0]
    cols = edge_index[1]

    a1d = _sc_spmm(x2, rows, cols, L_real_vals, L_imag_vals)
    a_full = a1d.reshape(2, C2, ACC_R)
    return _tc_matmul(a_full, X_real, X_imag, weight)
